# Initial kernel scaffold; baseline (speedup 1.0000x reference)
import jax
import jax.numpy as jnp
from jax.experimental import pallas as pl


def kernel(x, edge_index, W1, b1, W2, b2):
    raise NotImplementedError("write your pallas kernel here")


# speedup vs baseline: 6.9896x; 6.9896x over previous
import functools

import jax
import jax.numpy as jnp
from jax import lax
from jax.experimental import pallas as pl
from jax.experimental.pallas import tpu as pltpu
from jax.experimental.pallas import tpu_sc as plsc

N_NODES = 10000
N_EDGES = 320000
D_IN = 128
D_OUT = 16

NC = 2
NS = 16
NW = NC * NS
EPW = N_EDGES // NW
CHUNK = 80
NCHUNK = EPW // CHUNK
ROWS_MAIN = 632
ROWS_LAST = N_NODES - (NS - 1) * ROWS_MAIN


def _seg_sum_partials(src2d, dst2d, x, d):
    mesh = plsc.VectorSubcoreMesh(core_axis_name="c", subcore_axis_name="s")

    @functools.partial(
        pl.kernel,
        out_type=jax.ShapeDtypeStruct((NC, N_NODES, d), jnp.float32),
        mesh=mesh,
        scratch_types=[
            pltpu.VMEM((NCHUNK, CHUNK), jnp.int32),
            pltpu.VMEM((NCHUNK, CHUNK), jnp.int32),
            pltpu.VMEM((CHUNK, d), jnp.float32),
            pltpu.VMEM((8, d), jnp.float32),
            pltpu.VMEM_SHARED((N_NODES, d), jnp.float32),
            pltpu.SemaphoreType.DMA,
        ],
    )
    def body(src_hbm, dst_hbm, x_hbm, out_hbm, sidx, didx, rows, zbuf, acc, sem):
        cid = lax.axis_index("c")
        sid = lax.axis_index("s")
        wid = sid * NC + cid

        for i in range(8):
            for j in range(d // 16):
                zbuf[i, pl.ds(j * 16, 16)] = jnp.zeros((16,), jnp.float32)
        base = sid * ROWS_MAIN
        nrows = jnp.where(sid == NS - 1, ROWS_LAST, ROWS_MAIN)

        def zcopy(k, carry):
            pltpu.sync_copy(zbuf, acc.at[pl.ds(base + k * 8, 8)])
            return carry

        lax.fori_loop(0, nrows // 8, zcopy, 0)
        plsc.subcore_barrier()

        pltpu.sync_copy(src_hbm.at[wid], sidx)
        pltpu.sync_copy(dst_hbm.at[wid], didx)

        def step(i, carry):
            pltpu.async_copy(x_hbm.at[sidx.at[i]], rows, sem).wait()
            pltpu.sync_copy(rows, acc.at[didx.at[i]], add=True)
            return carry

        lax.fori_loop(0, NCHUNK, step, 0)

        plsc.subcore_barrier()

        @pl.when(sid < NS - 1)
        def _():
            pltpu.sync_copy(acc.at[pl.ds(base, ROWS_MAIN)],
                            out_hbm.at[cid, pl.ds(base, ROWS_MAIN)])

        @pl.when(sid == NS - 1)
        def _():
            pltpu.sync_copy(acc.at[pl.ds(base, ROWS_LAST)],
                            out_hbm.at[cid, pl.ds(base, ROWS_LAST)])

    return body(src2d, dst2d, x)


def _dense_mid(parts, W1, b1):
    R = 400

    def body(a_ref, w1_ref, b1_ref, o_ref):
        a = a_ref[0] + a_ref[1]
        h = jnp.dot(a, w1_ref[...], preferred_element_type=jnp.float32)
        o_ref[...] = jnp.maximum(h + b1_ref[...], 0.0)

    return pl.pallas_call(
        body,
        grid=(N_NODES // R,),
        in_specs=[
            pl.BlockSpec((NC, R, D_IN), lambda i: (0, i, 0)),
            pl.BlockSpec((D_IN, D_IN), lambda i: (0, 0)),
            pl.BlockSpec((1, D_IN), lambda i: (0, 0)),
        ],
        out_specs=pl.BlockSpec((R, D_IN), lambda i: (i, 0)),
        out_shape=jax.ShapeDtypeStruct((N_NODES, D_IN), jnp.float32),
    )(parts, W1, b1.reshape(1, D_IN))


def _final_logsoftmax(parts2, W2, b2):
    R = 400

    def body(a_ref, w2_ref, b2_ref, o_ref):
        a = a_ref[0] + a_ref[1]
        v = jnp.dot(a, w2_ref[...], preferred_element_type=jnp.float32)
        v = v + b2_ref[...]
        m = jnp.max(v, axis=1, keepdims=True)
        e = jnp.exp(v - m)
        s = jnp.sum(e, axis=1, keepdims=True)
        o_ref[...] = (v - m) - jnp.log(s)

    return pl.pallas_call(
        body,
        grid=(N_NODES // R,),
        in_specs=[
            pl.BlockSpec((NC, R, D_IN), lambda i: (0, i, 0)),
            pl.BlockSpec((D_IN, D_OUT), lambda i: (0, 0)),
            pl.BlockSpec((1, D_OUT), lambda i: (0, 0)),
        ],
        out_specs=pl.BlockSpec((R, D_OUT), lambda i: (i, 0)),
        out_shape=jax.ShapeDtypeStruct((N_NODES, D_OUT), jnp.float32),
    )(parts2, W2, b2.reshape(1, D_OUT))


def kernel(x, edge_index, W1, b1, W2, b2):
    src = edge_index[0].astype(jnp.int32).reshape(NW, NCHUNK, CHUNK)
    dst = edge_index[1].astype(jnp.int32).reshape(NW, NCHUNK, CHUNK)
    parts1 = _seg_sum_partials(src, dst, x, D_IN)
    h = _dense_mid(parts1, W1, b1)
    parts2 = _seg_sum_partials(src, dst, h, D_IN)
    return _final_logsoftmax(parts2, W2, b2)
